# baseline (speedup 1.0000x reference)
import functools

import jax
import jax.numpy as jnp
from jax.experimental import pallas as pl
from jax.experimental.pallas import tpu as pltpu


def _gcn_layer_kernel(adj_ref, x_ref, w_ref, b_ref, o_ref, s_ref, *, leaky):
    @pl.when(pl.program_id(1) == 0)
    def _():
        s_ref[...] = jnp.dot(
            x_ref[...], w_ref[...], preferred_element_type=jnp.float32
        )

    out = (
        jnp.dot(
            adj_ref[...].astype(jnp.bfloat16),
            s_ref[...].astype(jnp.bfloat16),
            preferred_element_type=jnp.float32,
        )
        + b_ref[...]
    )
    if leaky:
        out = jnp.where(out >= 0, out, 0.2 * out)
    o_ref[...] = out


def _row_tile(n):
    best = 8
    for bm in range(8, 513, 8):
        if n % bm == 0:
            best = bm
    return best


def _gcn_layer(adj, x, w, b, *, leaky):
    bsz, n, _ = adj.shape
    d = w.shape[1]
    bm = _row_tile(n)
    grid = (bsz, n // bm)

    if x.ndim == 2:
        x_spec = pl.BlockSpec((n, d), lambda bi, mi: (0, 0))
    else:
        x_spec = pl.BlockSpec((None, n, d), lambda bi, mi: (bi, 0, 0))

    return pl.pallas_call(
        functools.partial(_gcn_layer_kernel, leaky=leaky),
        grid=grid,
        in_specs=[
            pl.BlockSpec((None, bm, n), lambda bi, mi: (bi, mi, 0)),
            x_spec,
            pl.BlockSpec((d, d), lambda bi, mi: (0, 0)),
            pl.BlockSpec((1, d), lambda bi, mi: (0, 0)),
        ],
        out_specs=pl.BlockSpec((None, bm, d), lambda bi, mi: (bi, mi, 0)),
        out_shape=jax.ShapeDtypeStruct((bsz, n, d), jnp.float32),
        scratch_shapes=[pltpu.VMEM((n, d), jnp.float32)],
    )(adj, x, w, b)


@jax.jit
def kernel(batch, bx, W1, b1, W2, b2):
    b1 = b1.reshape(1, -1)
    b2 = b2.reshape(1, -1)
    x1 = _gcn_layer(batch, bx, W1, b1, leaky=True)
    out = _gcn_layer(batch, x1, W2, b2, leaky=False)
    return out


# speedup vs baseline: 1.0270x; 1.0270x over previous
import functools

import jax
import jax.numpy as jnp
from jax.experimental import pallas as pl
from jax.experimental.pallas import tpu as pltpu

_VMEM_BUDGET = 20 * 1024 * 1024


def _fused_kernel(
    adj_ref, bx_ref, w1_ref, b1_ref, w2_ref, b2_ref,
    o_ref, s_ref, x1_ref, *rest,
    bm, m, p, stride,
):
    pin_ref = rest[0] if p else None
    l = pl.program_id(1)
    i = pl.program_id(2)

    @pl.when((l == 0) & (i == 0))
    def _():
        s_ref[...] = jnp.dot(
            bx_ref[...], w1_ref[...], preferred_element_type=jnp.float32
        )

    @pl.when(l == 0)
    def _():
        tile = adj_ref[...]
        h = (
            jnp.dot(tile, s_ref[...], preferred_element_type=jnp.float32)
            + b1_ref[...]
        )
        x1_ref[pl.ds(i * bm, bm), :] = jnp.where(h >= 0, h, 0.2 * h)

        if p:
            @pl.when((i % stride == stride - 1) & (i < p * stride))
            def _():
                slot = jnp.minimum(i // stride, p - 1)
                pin_ref[slot] = tile

    @pl.when(l == 1)
    def _():
        @pl.when(i == 0)
        def _():
            s_ref[...] = jnp.dot(
                x1_ref[...], w2_ref[...], preferred_element_type=jnp.float32
            )

        j = m - 1 - i

        if p:
            pinned = (j % stride == stride - 1) & (j < p * stride)

            @pl.when(pinned)
            def _():
                slot = jnp.minimum(j // stride, p - 1)
                o_ref[...] = (
                    jnp.dot(
                        pin_ref[slot], s_ref[...],
                        preferred_element_type=jnp.float32,
                    )
                    + b2_ref[...]
                )

            not_pinned = jnp.logical_not(pinned)
        else:
            not_pinned = i >= 0

        @pl.when(not_pinned)
        def _():
            o_ref[...] = (
                jnp.dot(
                    adj_ref[...], s_ref[...],
                    preferred_element_type=jnp.float32,
                )
                + b2_ref[...]
            )


def _row_tile(n):
    best = 8
    for bm in range(8, 513, 8):
        if n % bm == 0:
            best = bm
    return best


@jax.jit
def kernel(batch, bx, W1, b1, W2, b2):
    bsz, n, _ = batch.shape
    d = bx.shape[1]
    bm = _row_tile(n)
    m = n // bm

    fixed = 2 * bm * n * 4 + 3 * n * d * 4 + 2 * bm * d * 4 + (1 << 20)
    p = max(0, (_VMEM_BUDGET - fixed) // (bm * n * 4))
    stride = m + 2
    if p > 0:
        stride = max(2, m // p)
        p = min(p, m // stride)

    b1 = b1.reshape(1, -1)
    b2 = b2.reshape(1, -1)

    def adj_index(b, l, i):
        j = m - 1 - i
        if p:
            pinned_j = (j % stride == stride - 1) & (j < p * stride)
            j = j + pinned_j.astype(j.dtype)
        return (b, jnp.where(l == 0, i, j), 0)

    def out_index(b, l, i):
        return (b, m - 1 - i * l, 0)

    const = lambda b, l, i: (0, 0)

    scratch = [
        pltpu.VMEM((n, d), jnp.float32),
        pltpu.VMEM((n, d), jnp.float32),
    ]
    if p:
        scratch.append(pltpu.VMEM((p, bm, n), jnp.float32))

    return pl.pallas_call(
        functools.partial(_fused_kernel, bm=bm, m=m, p=p, stride=stride),
        grid=(bsz, 2, m),
        in_specs=[
            pl.BlockSpec((None, bm, n), adj_index),
            pl.BlockSpec((n, d), const),
            pl.BlockSpec((d, d), const),
            pl.BlockSpec((1, d), const),
            pl.BlockSpec((d, d), const),
            pl.BlockSpec((1, d), const),
        ],
        out_specs=pl.BlockSpec((None, bm, d), out_index),
        out_shape=jax.ShapeDtypeStruct((bsz, n, d), jnp.float32),
        scratch_shapes=scratch,
        compiler_params=pltpu.CompilerParams(
            dimension_semantics=("parallel", "arbitrary", "arbitrary"),
        ),
    )(batch, bx, W1, b1, W2, b2)
